# baseline (speedup 1.0000x reference)
import functools

import jax
import jax.numpy as jnp
from jax import lax
from jax.experimental import pallas as pl
from jax.experimental.pallas import tpu as pltpu
from jax.experimental.pallas import tpu_sc as plsc

_QP = 15
_CHUNK = 32768


def _make_sc_call(n, nc, ns, L, per_w, chunk, nchunks, dtype):
    mesh = plsc.VectorSubcoreMesh(
        core_axis_name="c", subcore_axis_name="s", num_cores=nc, num_subcores=ns
    )

    @functools.partial(
        pl.kernel,
        out_type=jax.ShapeDtypeStruct((n,), dtype),
        mesh=mesh,
        compiler_params=pltpu.CompilerParams(needs_layout_passes=False),
        scratch_types=[
            pltpu.VMEM((L,), jnp.float32),
            pltpu.VMEM((L,), jnp.float32),
            pltpu.VMEM((chunk,), jnp.float32),
            pltpu.VMEM((chunk,), jnp.float32),
        ],
    )
    def run(x_hbm, s_hbm, o_hbm, btab, ltab, ibuf, obuf):
        wid = lax.axis_index("s") * nc + lax.axis_index("c")
        base = wid * per_w

        pltpu.sync_copy(s_hbm, btab)
        sv = btab[...]
        iota = lax.broadcasted_iota(jnp.int32, (L,), 0)
        cs = sv
        for d in (1, 2, 4, 8):
            ltab[...] = cs
            g = plsc.load_gather(ltab, [jnp.maximum(iota - d, 0)])
            cs = cs + jnp.where(iota >= d, g, jnp.float32(0.0))
        ltab[...] = cs
        lv = plsc.load_gather(ltab, [jnp.maximum(iota - 1, 0)])
        lv = jnp.where(iota == 0, jnp.float32(0.0), lv)
        btab[...] = cs - sv * 0.5
        ltab[...] = lv

        def chunk_body(c, carry):
            off = base + c * chunk
            pltpu.sync_copy(x_hbm.at[pl.ds(off, chunk)], ibuf)

            @plsc.parallel_loop(0, chunk // L, unroll=8)
            def inner(i):
                xv = ibuf[pl.ds(i * L, L)]
                idx = jnp.zeros((L,), jnp.int32)
                for step in (8, 4, 2, 1):
                    bv = plsc.load_gather(btab, [idx + (step - 1)])
                    idx = idx + jnp.where(bv < xv, step, 0)
                obuf[pl.ds(i * L, L)] = plsc.load_gather(ltab, [idx])

            pltpu.sync_copy(obuf, o_hbm.at[pl.ds(off, chunk)])
            return carry

        lax.fori_loop(0, nchunks, chunk_body, 0)

    return run


def kernel(x, scale, Qn, Qp, num_elements, box_size):
    info = plsc.get_sparse_core_info()
    NC, NS, L = info.num_cores, info.num_subcores, info.num_lanes
    nw = NC * NS
    n = x.size
    xf = x.reshape(n)
    scale16 = jnp.zeros((L,), x.dtype).at[: scale.shape[0]].set(scale)
    per_w = n // nw
    chunk = min(_CHUNK, per_w)
    nchunks = per_w // chunk
    run = _make_sc_call(n, NC, NS, L, per_w, chunk, nchunks, x.dtype)
    y = run(xf, scale16)
    return y.reshape(x.shape)


# speedup vs baseline: 1.2376x; 1.2376x over previous
import functools

import jax
import jax.numpy as jnp
from jax import lax
from jax.experimental import pallas as pl
from jax.experimental.pallas import tpu as pltpu
from jax.experimental.pallas import tpu_sc as plsc

_QP = 15
_CHUNK = 16384


def _make_sc_call(n, nc, ns, L, per_w, chunk, nchunks, dtype):
    mesh = plsc.VectorSubcoreMesh(
        core_axis_name="c", subcore_axis_name="s", num_cores=nc, num_subcores=ns
    )

    @functools.partial(
        pl.kernel,
        out_type=jax.ShapeDtypeStruct((n,), dtype),
        mesh=mesh,
        compiler_params=pltpu.CompilerParams(needs_layout_passes=False),
        scratch_types=[
            pltpu.VMEM((L,), jnp.float32),
            pltpu.VMEM((L,), jnp.float32),
            pltpu.VMEM((chunk,), jnp.float32),
            pltpu.VMEM((chunk,), jnp.float32),
            pltpu.VMEM((chunk,), jnp.float32),
            pltpu.VMEM((chunk,), jnp.float32),
            pltpu.SemaphoreType.DMA,
            pltpu.SemaphoreType.DMA,
            pltpu.SemaphoreType.DMA,
            pltpu.SemaphoreType.DMA,
        ],
    )
    def run(x_hbm, s_hbm, o_hbm, btab, ltab, ib0, ib1, ob0, ob1, si0, si1, so0, so1):
        wid = lax.axis_index("s") * nc + lax.axis_index("c")
        base = wid * per_w

        pltpu.sync_copy(s_hbm, btab)
        sv = btab[...]
        iota = lax.broadcasted_iota(jnp.int32, (L,), 0)
        cs = sv
        for d in (1, 2, 4, 8):
            ltab[...] = cs
            g = plsc.load_gather(ltab, [jnp.maximum(iota - d, 0)])
            cs = cs + jnp.where(iota >= d, g, jnp.float32(0.0))
        ltab[...] = cs
        lv = plsc.load_gather(ltab, [jnp.maximum(iota - 1, 0)])
        lv = jnp.where(iota == 0, jnp.float32(0.0), lv)
        btab[...] = cs - sv * 0.5
        ltab[...] = lv

        def compute(ibuf, obuf):
            @plsc.parallel_loop(0, chunk // L, unroll=8)
            def inner(i):
                xv = ibuf[pl.ds(i * L, L)]
                idx = jnp.zeros((L,), jnp.int32)
                for step in (8, 4, 2, 1):
                    bv = plsc.load_gather(btab, [idx + (step - 1)])
                    idx = idx + jnp.where(bv < xv, step, 0)
                obuf[pl.ds(i * L, L)] = plsc.load_gather(ltab, [idx])

        def in_slice(c):
            return x_hbm.at[pl.ds(base + c * chunk, chunk)]

        def out_slice(c):
            return o_hbm.at[pl.ds(base + c * chunk, chunk)]

        npairs = nchunks // 2
        pltpu.async_copy(in_slice(0), ib0, si0)

        def pair_body(p, carry):
            c0 = 2 * p
            pltpu.async_copy(in_slice(c0 + 1), ib1, si1)
            pltpu.make_async_copy(in_slice(c0), ib0, si0).wait()

            @pl.when(p > 0)
            def _():
                pltpu.make_async_copy(ob0, out_slice(c0), so0).wait()

            compute(ib0, ob0)
            pltpu.async_copy(ob0, out_slice(c0), so0)

            @pl.when(p + 1 < npairs)
            def _():
                pltpu.async_copy(in_slice(c0 + 2), ib0, si0)

            pltpu.make_async_copy(in_slice(c0 + 1), ib1, si1).wait()

            @pl.when(p > 0)
            def _():
                pltpu.make_async_copy(ob1, out_slice(c0 + 1), so1).wait()

            compute(ib1, ob1)
            pltpu.async_copy(ob1, out_slice(c0 + 1), so1)
            return carry

        lax.fori_loop(0, npairs, pair_body, 0)
        pltpu.make_async_copy(ob0, out_slice(nchunks - 2), so0).wait()
        pltpu.make_async_copy(ob1, out_slice(nchunks - 1), so1).wait()

    return run


def kernel(x, scale, Qn, Qp, num_elements, box_size):
    info = plsc.get_sparse_core_info()
    NC, NS, L = info.num_cores, info.num_subcores, info.num_lanes
    nw = NC * NS
    n = x.size
    xf = x.reshape(n)
    scale16 = jnp.zeros((L,), x.dtype).at[: scale.shape[0]].set(scale)
    per_w = n // nw
    chunk = min(_CHUNK, per_w)
    nchunks = per_w // chunk
    run = _make_sc_call(n, NC, NS, L, per_w, chunk, nchunks, x.dtype)
    y = run(xf, scale16)
    return y.reshape(x.shape)
